# baseline (speedup 1.0000x reference)
import functools

import jax
import jax.numpy as jnp
from jax.experimental import pallas as pl

_MIN_NORM = 1e-15
_ROW_BLOCK = 10000


def _decoder_block(x_ref, w_ref, b_ref, o_ref):
    x = x_ref[...]
    sq = jnp.sum(x * x, axis=1, keepdims=True)
    norm = jnp.sqrt(sq)
    p_norm = jnp.maximum(norm, _MIN_NORM)
    t = jnp.clip(norm, -1.0 + 1e-7, 1.0 - 1e-7)
    scale = 0.5 * jnp.log((1.0 + t) / (1.0 - t)) / p_norm
    y = jax.lax.dot_general(
        x, w_ref[...],
        dimension_numbers=(((1,), (1,)), ((), ())),
        preferred_element_type=jnp.float32,
    )
    o_ref[...] = scale * y + b_ref[...]


@functools.partial(jax.jit, static_argnames=())
def kernel(x, adj, W, b):
    del adj
    n, d = x.shape
    c = W.shape[0]
    grid = (n // _ROW_BLOCK,)
    return pl.pallas_call(
        _decoder_block,
        grid=grid,
        in_specs=[
            pl.BlockSpec((_ROW_BLOCK, d), lambda i: (i, 0)),
            pl.BlockSpec((c, d), lambda i: (0, 0)),
            pl.BlockSpec((1, c), lambda i: (0, 0)),
        ],
        out_specs=pl.BlockSpec((_ROW_BLOCK, c), lambda i: (i, 0)),
        out_shape=jax.ShapeDtypeStruct((n, c), jnp.float32),
    )(x, W, b[None, :])


# speedup vs baseline: 1.0089x; 1.0089x over previous
import functools

import jax
import jax.numpy as jnp
from jax.experimental import pallas as pl

_ROW_BLOCK = 5000


def _decoder_block(x_ref, wt_ref, b_ref, o_ref):
    x = x_ref[...]
    x2 = x * x
    ones = jnp.ones((x.shape[1], 1), dtype=jnp.float32)
    sq = jax.lax.dot_general(
        x2, ones,
        dimension_numbers=(((1,), (0,)), ((), ())),
        preferred_element_type=jnp.float32,
    )
    inv_norm = jax.lax.rsqrt(jnp.maximum(sq, 1e-30))
    t = jnp.minimum(sq * inv_norm, 1.0 - 1e-7)
    scale = (0.5 * inv_norm) * jnp.log((1.0 + t) / (1.0 - t))
    y = jax.lax.dot_general(
        x, wt_ref[...],
        dimension_numbers=(((1,), (0,)), ((), ())),
        preferred_element_type=jnp.float32,
    )
    o_ref[...] = scale * y + b_ref[...]


@functools.partial(jax.jit, static_argnames=())
def kernel(x, adj, W, b):
    del adj
    n, d = x.shape
    c = W.shape[0]
    grid = (n // _ROW_BLOCK,)
    return pl.pallas_call(
        _decoder_block,
        grid=grid,
        in_specs=[
            pl.BlockSpec((_ROW_BLOCK, d), lambda i: (i, 0)),
            pl.BlockSpec((d, c), lambda i: (0, 0)),
            pl.BlockSpec((1, c), lambda i: (0, 0)),
        ],
        out_specs=pl.BlockSpec((_ROW_BLOCK, c), lambda i: (i, 0)),
        out_shape=jax.ShapeDtypeStruct((n, c), jnp.float32),
    )(x, W.T, b[None, :])


# speedup vs baseline: 1.1371x; 1.1271x over previous
import functools

import jax
import jax.numpy as jnp
from jax.experimental import pallas as pl

_ROW_BLOCK = 5000


def _decoder_block(x_ref, wt_ref, b_ref, o_ref):
    x = x_ref[...]
    x2 = x * x
    ones = jnp.ones((x.shape[1], 1), dtype=jnp.float32)
    sq = jax.lax.dot_general(
        x2, ones,
        dimension_numbers=(((1,), (0,)), ((), ())),
        preferred_element_type=jnp.float32,
    )
    inv_norm = jax.lax.rsqrt(jnp.maximum(sq, 1e-30))
    t = jnp.minimum(sq * inv_norm, 1.0 - 1e-7)
    scale = (0.5 * inv_norm) * jnp.log((1.0 + t) / (1.0 - t))
    y = jax.lax.dot_general(
        x, wt_ref[...],
        dimension_numbers=(((1,), (1,)), ((), ())),
        preferred_element_type=jnp.float32,
    )
    o_ref[...] = scale * y + b_ref[...]


@functools.partial(jax.jit, static_argnames=())
def kernel(x, adj, W, b):
    del adj
    n, d = x.shape
    c = W.shape[0]
    grid = (n // _ROW_BLOCK,)
    return pl.pallas_call(
        _decoder_block,
        grid=grid,
        in_specs=[
            pl.BlockSpec((_ROW_BLOCK, d), lambda i: (i, 0)),
            pl.BlockSpec((c, d), lambda i: (0, 0)),
            pl.BlockSpec((1, c), lambda i: (0, 0)),
        ],
        out_specs=pl.BlockSpec((_ROW_BLOCK, c), lambda i: (i, 0)),
        out_shape=jax.ShapeDtypeStruct((n, c), jnp.float32),
    )(x, W, b[None, :])
